# baseline (speedup 1.0000x reference)
import functools

import jax
import jax.numpy as jnp
from jax import lax
from jax.experimental import pallas as pl
from jax.experimental.pallas import tpu as pltpu
from jax.experimental.pallas import tpu_sc as plsc


def _sc_permute(values, weights, W, C):
    info = plsc.get_sparse_core_info()
    nc, ns = info.num_cores, info.num_subcores
    nw = nc * ns
    pairs = W * W
    per_w = pairs // nw

    mesh = plsc.VectorSubcoreMesh(core_axis_name="c", subcore_axis_name="s")

    @functools.partial(
        pl.kernel,
        mesh=mesh,
        out_type=[
            jax.ShapeDtypeStruct((W, W * C), jnp.int32),
            jax.ShapeDtypeStruct((W, W * C), jnp.float32),
        ],
    )
    def k(vals_hbm, wts_hbm, out_i_hbm, out_w_hbm):
        wid = lax.axis_index("s") * nc + lax.axis_index("c")
        for t in range(per_w):
            p = wid * per_w + t
            s = p // W
            w = p % W
            pltpu.sync_copy(vals_hbm.at[s, w], out_i_hbm.at[w, pl.ds(s * C, C)])
            pltpu.sync_copy(wts_hbm.at[s, w], out_w_hbm.at[w, pl.ds(s * C, C)])

    return k(values, weights)


def _tc_constants(W, N):
    def body(len_ref, off_ref):
        len_ref[...] = jnp.ones(len_ref.shape, jnp.int32)
        off_ref[...] = lax.broadcasted_iota(jnp.int32, off_ref.shape, 2)

    lengths3, offsets3 = pl.pallas_call(
        body,
        grid=(W,),
        out_specs=[
            pl.BlockSpec((1, 1, N), lambda i: (i, 0, 0)),
            pl.BlockSpec((1, 1, N + 1), lambda i: (i, 0, 0)),
        ],
        out_shape=[
            jax.ShapeDtypeStruct((W, 1, N), jnp.int32),
            jax.ShapeDtypeStruct((W, 1, N + 1), jnp.int32),
        ],
    )()
    return lengths3.reshape(W, N), offsets3.reshape(W, N + 1)


def kernel(values, weights):
    W, _, C = values.shape
    N = W * C
    out_indices, out_weights = _sc_permute(values, weights, W, C)
    kjt_lengths, kjt_offsets = _tc_constants(W, N)
    return out_indices, out_weights, kjt_lengths, kjt_offsets


# speedup vs baseline: 15.7403x; 15.7403x over previous
import functools

import jax
import jax.numpy as jnp
from jax import lax
from jax.experimental import pallas as pl
from jax.experimental.pallas import tpu as pltpu
from jax.experimental.pallas import tpu_sc as plsc


_PIECE = 16384
_NB = 2


def _sc_permute(values, weights, W, C):
    info = plsc.get_sparse_core_info()
    nc, ns = info.num_cores, info.num_subcores
    nw = nc * ns
    pairs = W * W
    per_w = pairs // nw
    npieces = C // _PIECE

    mesh = plsc.VectorSubcoreMesh(core_axis_name="c", subcore_axis_name="s")

    @functools.partial(
        pl.kernel,
        mesh=mesh,
        out_type=[
            jax.ShapeDtypeStruct((W, W * C), jnp.int32),
            jax.ShapeDtypeStruct((W, W * C), jnp.float32),
        ],
        scratch_types=[
            pltpu.VMEM((_NB, _PIECE), jnp.int32),
            pltpu.VMEM((_NB, _PIECE), jnp.float32),
            pltpu.SemaphoreType.DMA,
            pltpu.SemaphoreType.DMA,
            pltpu.SemaphoreType.DMA,
            pltpu.SemaphoreType.DMA,
        ],
    )
    def k(vals_hbm, wts_hbm, out_i_hbm, out_w_hbm, vbuf, wbuf,
          sin0, sin1, sout0, sout1):
        sins = (sin0, sin1)
        souts = (sout0, sout1)
        wid = lax.axis_index("s") * nc + lax.axis_index("c")

        def stream_chunk(src, dst, buf, s, w):
            out_h = [None] * _NB
            for j in range(npieces):
                b = j % _NB
                if out_h[b] is not None:
                    out_h[b].wait()
                pltpu.async_copy(
                    src.at[s, w, pl.ds(j * _PIECE, _PIECE)], buf.at[b], sins[b]
                ).wait()
                out_h[b] = pltpu.async_copy(
                    buf.at[b],
                    dst.at[w, pl.ds(s * C + j * _PIECE, _PIECE)],
                    souts[b],
                )
            for b in range(_NB):
                if out_h[b] is not None:
                    out_h[b].wait()

        for t in range(per_w):
            p = wid * per_w + t
            s = p // W
            w = p % W
            stream_chunk(vals_hbm, out_i_hbm, vbuf, s, w)
            stream_chunk(wts_hbm, out_w_hbm, wbuf, s, w)

    return k(values, weights)


def _tc_constants(W, N):
    def body(len_ref, off_ref):
        len_ref[...] = jnp.ones(len_ref.shape, jnp.int32)
        off_ref[...] = lax.broadcasted_iota(jnp.int32, off_ref.shape, 2)

    lengths3, offsets3 = pl.pallas_call(
        body,
        grid=(W,),
        out_specs=[
            pl.BlockSpec((1, 1, N), lambda i: (i, 0, 0)),
            pl.BlockSpec((1, 1, N + 1), lambda i: (i, 0, 0)),
        ],
        out_shape=[
            jax.ShapeDtypeStruct((W, 1, N), jnp.int32),
            jax.ShapeDtypeStruct((W, 1, N + 1), jnp.int32),
        ],
    )()
    return lengths3.reshape(W, N), offsets3.reshape(W, N + 1)


def kernel(values, weights):
    W, _, C = values.shape
    N = W * C
    out_indices, out_weights = _sc_permute(values, weights, W, C)
    kjt_lengths, kjt_offsets = _tc_constants(W, N)
    return out_indices, out_weights, kjt_lengths, kjt_offsets
